# baseline (speedup 1.0000x reference)
import functools

import jax
import jax.numpy as jnp
from jax.experimental import pallas as pl
from jax.experimental.pallas import tpu as pltpu

_B, _S, _D, _F, _C, _R = 2, 2048, 1024, 4096, 8, 4
_CR = _C * _R
_ALPHA = 1.0

_S_TILE = 4096
_F_TILE = 512


def _mlp_lora_kernel(logits_ref, e_ref, x_ref, w1_ref, b1_ref, lb1_ref,
                     a1_ref, b1f_ref, w2_ref, b2_ref, lb2_ref, a2_ref,
                     b2f_ref, out_ref):
    j = pl.program_id(1)

    logits = logits_ref[...]
    m = jnp.max(logits, axis=1, keepdims=True)
    ex = jnp.exp(logits - m)
    probs = ex / jnp.sum(ex, axis=1, keepdims=True)
    pr = jnp.dot(probs, e_ref[...], preferred_element_type=jnp.float32)

    w1_eff = (w1_ref[...] + _ALPHA * jnp.dot(
        b1f_ref[...] * pr, a1_ref[...],
        preferred_element_type=jnp.float32)).astype(jnp.bfloat16)
    bias1 = b1_ref[...] + _ALPHA * jnp.dot(
        probs, lb1_ref[...], preferred_element_type=jnp.float32)

    h = jax.lax.dot_general(
        x_ref[...], w1_eff, (((1,), (1,)), ((), ())),
        preferred_element_type=jnp.float32)
    h = jnp.maximum(h + bias1, 0.0).astype(jnp.bfloat16)

    w2_eff = (w2_ref[...] + _ALPHA * jnp.dot(
        b2f_ref[...] * pr, a2_ref[...],
        preferred_element_type=jnp.float32)).astype(jnp.bfloat16)

    contrib = jax.lax.dot_general(
        h, w2_eff, (((1,), (1,)), ((), ())),
        preferred_element_type=jnp.float32)

    @pl.when(j == 0)
    def _init():
        bias2 = b2_ref[...] + _ALPHA * jnp.dot(
            probs, lb2_ref[...], preferred_element_type=jnp.float32)
        out_ref[...] = contrib + bias2

    @pl.when(j != 0)
    def _acc():
        out_ref[...] += contrib


@functools.partial(jax.jit, static_argnames=())
def kernel(x, W1, b1, W2, b2, router_logits, A1, B1, lb1, A2, B2, lb2):
    bsz, seq, d = x.shape
    n_tok = bsz * seq
    xf = x.reshape(n_tok, d).astype(jnp.bfloat16)

    a1f = A1.reshape(_CR, _D)
    b1f = B1.transpose(1, 0, 2).reshape(_F, _CR)
    a2f = A2.reshape(_CR, _F)
    b2f = B2.transpose(1, 0, 2).reshape(_D, _CR)

    logits2d = router_logits.reshape(1, _C)
    b1r = b1.reshape(1, _F)
    b2r = b2.reshape(1, _D)

    e = (jax.lax.broadcasted_iota(jnp.int32, (_C, _CR), 1) // _R ==
         jax.lax.broadcasted_iota(jnp.int32, (_C, _CR), 0)).astype(jnp.float32)

    n_s = n_tok // _S_TILE
    n_f = _F // _F_TILE

    out = pl.pallas_call(
        _mlp_lora_kernel,
        grid=(n_s, n_f),
        in_specs=[
            pl.BlockSpec((1, _C), lambda i, j: (0, 0)),
            pl.BlockSpec((_C, _CR), lambda i, j: (0, 0)),
            pl.BlockSpec((_S_TILE, _D), lambda i, j: (i, 0)),
            pl.BlockSpec((_F_TILE, _D), lambda i, j: (j, 0)),
            pl.BlockSpec((1, _F_TILE), lambda i, j: (0, j)),
            pl.BlockSpec((_C, _F_TILE), lambda i, j: (0, j)),
            pl.BlockSpec((_CR, _D), lambda i, j: (0, 0)),
            pl.BlockSpec((_F_TILE, _CR), lambda i, j: (j, 0)),
            pl.BlockSpec((_D, _F_TILE), lambda i, j: (0, j)),
            pl.BlockSpec((1, _D), lambda i, j: (0, 0)),
            pl.BlockSpec((_C, _D), lambda i, j: (0, 0)),
            pl.BlockSpec((_CR, _F_TILE), lambda i, j: (0, j)),
            pl.BlockSpec((_D, _CR), lambda i, j: (0, 0)),
        ],
        out_specs=pl.BlockSpec((_S_TILE, _D), lambda i, j: (i, 0)),
        out_shape=jax.ShapeDtypeStruct((n_tok, _D), jnp.float32),
        compiler_params=pltpu.CompilerParams(
            vmem_limit_bytes=100 * 1024 * 1024),
    )(logits2d, e, xf, W1, b1r, lb1, a1f, b1f, W2, b2r, lb2, a2f, b2f)

    return out.reshape(bsz, seq, d)


# speedup vs baseline: 1.1121x; 1.1121x over previous
import functools

import jax
import jax.numpy as jnp
from jax.experimental import pallas as pl
from jax.experimental.pallas import tpu as pltpu

_B, _S, _D, _F, _C, _R = 2, 2048, 1024, 4096, 8, 4
_CR = _C * _R
_ALPHA = 1.0

_N_TOK = _B * _S
_F_TILE = 512
_S_TILE_B = 1024


def _softmax_pr(logits_ref, e_ref):
    logits = logits_ref[...]
    m = jnp.max(logits, axis=1, keepdims=True)
    ex = jnp.exp(logits - m)
    probs = ex / jnp.sum(ex, axis=1, keepdims=True)
    pr = jnp.dot(probs, e_ref[...], preferred_element_type=jnp.float32)
    return probs, pr


def _layer1_kernel(logits_ref, e_ref, x_ref, w1_ref, b1_ref, lb1_ref,
                   a1_ref, b1f_ref, h_ref):
    probs, pr = _softmax_pr(logits_ref, e_ref)
    w1_eff = w1_ref[...] + _ALPHA * jnp.dot(
        b1f_ref[...] * pr, a1_ref[...], preferred_element_type=jnp.float32)
    bias1 = b1_ref[...] + _ALPHA * jnp.dot(
        probs, lb1_ref[...], preferred_element_type=jnp.float32)
    h = jax.lax.dot_general(
        x_ref[...], w1_eff, (((1,), (1,)), ((), ())),
        preferred_element_type=jnp.float32)
    h_ref[...] = jnp.maximum(h + bias1, 0.0).astype(jnp.bfloat16)


def _layer2_kernel(logits_ref, e_ref, h_ref, w2_ref, b2_ref, lb2_ref,
                   a2_ref, b2f_ref, out_ref, w2eff_ref):
    i = pl.program_id(0)
    probs, pr = _softmax_pr(logits_ref, e_ref)

    @pl.when(i == 0)
    def _fold():
        w2eff_ref[...] = (w2_ref[...] + _ALPHA * jnp.dot(
            b2f_ref[...] * pr, a2_ref[...],
            preferred_element_type=jnp.float32)).astype(jnp.bfloat16)

    bias2 = b2_ref[...] + _ALPHA * jnp.dot(
        probs, lb2_ref[...], preferred_element_type=jnp.float32)
    acc = jax.lax.dot_general(
        h_ref[...], w2eff_ref[...], (((1,), (1,)), ((), ())),
        preferred_element_type=jnp.float32)
    out_ref[...] = acc + bias2


@functools.partial(jax.jit, static_argnames=())
def kernel(x, W1, b1, W2, b2, router_logits, A1, B1, lb1, A2, B2, lb2):
    bsz, seq, d = x.shape
    xf = x.reshape(_N_TOK, d)

    a1f = A1.reshape(_CR, _D)
    b1f = B1.transpose(1, 0, 2).reshape(_F, _CR)
    a2f = A2.reshape(_CR, _F)
    b2f = B2.transpose(1, 0, 2).reshape(_D, _CR)

    logits2d = router_logits.reshape(1, _C)
    b1r = b1.reshape(1, _F)
    b2r = b2.reshape(1, _D)

    e = (jax.lax.broadcasted_iota(jnp.int32, (_C, _CR), 1) // _R ==
         jax.lax.broadcasted_iota(jnp.int32, (_C, _CR), 0)).astype(jnp.float32)

    n_f = _F // _F_TILE
    h = pl.pallas_call(
        _layer1_kernel,
        grid=(n_f,),
        in_specs=[
            pl.BlockSpec((1, _C), lambda j: (0, 0)),
            pl.BlockSpec((_C, _CR), lambda j: (0, 0)),
            pl.BlockSpec((_N_TOK, _D), lambda j: (0, 0)),
            pl.BlockSpec((_F_TILE, _D), lambda j: (j, 0)),
            pl.BlockSpec((1, _F_TILE), lambda j: (0, j)),
            pl.BlockSpec((_C, _F_TILE), lambda j: (0, j)),
            pl.BlockSpec((_CR, _D), lambda j: (0, 0)),
            pl.BlockSpec((_F_TILE, _CR), lambda j: (j, 0)),
        ],
        out_specs=pl.BlockSpec((_N_TOK, _F_TILE), lambda j: (0, j)),
        out_shape=jax.ShapeDtypeStruct((_N_TOK, _F), jnp.bfloat16),
        compiler_params=pltpu.CompilerParams(
            vmem_limit_bytes=100 * 1024 * 1024),
    )(logits2d, e, xf, W1, b1r, lb1, a1f, b1f)

    n_s = _N_TOK // _S_TILE_B
    out = pl.pallas_call(
        _layer2_kernel,
        grid=(n_s,),
        in_specs=[
            pl.BlockSpec((1, _C), lambda i: (0, 0)),
            pl.BlockSpec((_C, _CR), lambda i: (0, 0)),
            pl.BlockSpec((_S_TILE_B, _F), lambda i: (i, 0)),
            pl.BlockSpec((_D, _F), lambda i: (0, 0)),
            pl.BlockSpec((1, _D), lambda i: (0, 0)),
            pl.BlockSpec((_C, _D), lambda i: (0, 0)),
            pl.BlockSpec((_CR, _F), lambda i: (0, 0)),
            pl.BlockSpec((_D, _CR), lambda i: (0, 0)),
        ],
        out_specs=pl.BlockSpec((_S_TILE_B, _D), lambda i: (i, 0)),
        out_shape=jax.ShapeDtypeStruct((_N_TOK, _D), jnp.float32),
        scratch_shapes=[pltpu.VMEM((_D, _F), jnp.bfloat16)],
        compiler_params=pltpu.CompilerParams(
            vmem_limit_bytes=100 * 1024 * 1024),
    )(logits2d, e, h, W2, b2r, lb2, a2f, b2f)

    return out.reshape(bsz, seq, d)


# speedup vs baseline: 1.1315x; 1.0174x over previous
import functools

import jax
import jax.numpy as jnp
from jax.experimental import pallas as pl
from jax.experimental.pallas import tpu as pltpu

_B, _S, _D, _F, _C, _R = 2, 2048, 1024, 4096, 8, 4
_CR = _C * _R
_ALPHA = 1.0

_N_TOK = _B * _S
_F_TILE = 512
_S_TILE_B = 1024


def _softmax_pr(logits_ref, e_ref):
    logits = logits_ref[...]
    m = jnp.max(logits, axis=1, keepdims=True)
    ex = jnp.exp(logits - m)
    probs = ex / jnp.sum(ex, axis=1, keepdims=True)
    pr = jnp.dot(probs, e_ref[...], preferred_element_type=jnp.float32)
    return probs, pr


def _layer1_kernel(logits_ref, e_ref, x_ref, w1_ref, b1_ref, lb1_ref,
                   a1_ref, b1f_ref, w2_ref, a2_ref, b2f_ref,
                   h_ref, w2eff_ref):
    probs, pr = _softmax_pr(logits_ref, e_ref)
    w1_eff = w1_ref[...] + _ALPHA * jnp.dot(
        b1f_ref[...] * pr, a1_ref[...], preferred_element_type=jnp.float32)
    bias1 = b1_ref[...] + _ALPHA * jnp.dot(
        probs, lb1_ref[...], preferred_element_type=jnp.float32)
    h = jax.lax.dot_general(
        x_ref[...], w1_eff, (((1,), (1,)), ((), ())),
        preferred_element_type=jnp.float32)
    h_ref[...] = jnp.maximum(h + bias1, 0.0).astype(jnp.bfloat16)
    w2eff_ref[...] = (w2_ref[...] + _ALPHA * jnp.dot(
        b2f_ref[...] * pr, a2_ref[...],
        preferred_element_type=jnp.float32)).astype(jnp.bfloat16)


def _layer2_kernel(logits_ref, e_ref, h_ref, w2eff_ref, b2_ref, lb2_ref,
                   out_ref):
    probs, _ = _softmax_pr(logits_ref, e_ref)
    bias2 = b2_ref[...] + _ALPHA * jnp.dot(
        probs, lb2_ref[...], preferred_element_type=jnp.float32)
    acc = jax.lax.dot_general(
        h_ref[...], w2eff_ref[...], (((1,), (1,)), ((), ())),
        preferred_element_type=jnp.float32)
    out_ref[...] = acc + bias2


@functools.partial(jax.jit, static_argnames=())
def kernel(x, W1, b1, W2, b2, router_logits, A1, B1, lb1, A2, B2, lb2):
    bsz, seq, d = x.shape
    xf = x.reshape(_N_TOK, d)

    a1f = A1.reshape(_CR, _D)
    b1f = B1.transpose(1, 0, 2).reshape(_F, _CR)
    a2f = A2.reshape(_CR, _F)
    b2f = B2.transpose(1, 0, 2).reshape(_D, _CR)

    logits2d = router_logits.reshape(1, _C)
    b1r = b1.reshape(1, _F)
    b2r = b2.reshape(1, _D)

    e = (jax.lax.broadcasted_iota(jnp.int32, (_C, _CR), 1) // _R ==
         jax.lax.broadcasted_iota(jnp.int32, (_C, _CR), 0)).astype(jnp.float32)

    n_f = _F // _F_TILE
    h, w2eff = pl.pallas_call(
        _layer1_kernel,
        grid=(n_f,),
        in_specs=[
            pl.BlockSpec((1, _C), lambda j: (0, 0)),
            pl.BlockSpec((_C, _CR), lambda j: (0, 0)),
            pl.BlockSpec((_N_TOK, _D), lambda j: (0, 0)),
            pl.BlockSpec((_F_TILE, _D), lambda j: (j, 0)),
            pl.BlockSpec((1, _F_TILE), lambda j: (0, j)),
            pl.BlockSpec((_C, _F_TILE), lambda j: (0, j)),
            pl.BlockSpec((_CR, _D), lambda j: (0, 0)),
            pl.BlockSpec((_F_TILE, _CR), lambda j: (j, 0)),
            pl.BlockSpec((_D, _F_TILE), lambda j: (0, j)),
            pl.BlockSpec((_CR, _F_TILE), lambda j: (0, j)),
            pl.BlockSpec((_D, _CR), lambda j: (0, 0)),
        ],
        out_specs=[
            pl.BlockSpec((_N_TOK, _F_TILE), lambda j: (0, j)),
            pl.BlockSpec((_D, _F_TILE), lambda j: (0, j)),
        ],
        out_shape=[
            jax.ShapeDtypeStruct((_N_TOK, _F), jnp.bfloat16),
            jax.ShapeDtypeStruct((_D, _F), jnp.bfloat16),
        ],
        compiler_params=pltpu.CompilerParams(
            vmem_limit_bytes=100 * 1024 * 1024),
    )(logits2d, e, xf, W1, b1r, lb1, a1f, b1f, W2, a2f, b2f)

    n_s = _N_TOK // _S_TILE_B
    out = pl.pallas_call(
        _layer2_kernel,
        grid=(n_s,),
        in_specs=[
            pl.BlockSpec((1, _C), lambda i: (0, 0)),
            pl.BlockSpec((_C, _CR), lambda i: (0, 0)),
            pl.BlockSpec((_S_TILE_B, _F), lambda i: (i, 0)),
            pl.BlockSpec((_D, _F), lambda i: (0, 0)),
            pl.BlockSpec((1, _D), lambda i: (0, 0)),
            pl.BlockSpec((_C, _D), lambda i: (0, 0)),
        ],
        out_specs=pl.BlockSpec((_S_TILE_B, _D), lambda i: (i, 0)),
        out_shape=jax.ShapeDtypeStruct((_N_TOK, _D), jnp.float32),
        compiler_params=pltpu.CompilerParams(
            vmem_limit_bytes=100 * 1024 * 1024),
    )(logits2d, e, h, w2eff, b2r, lb2)

    return out.reshape(bsz, seq, d)


# speedup vs baseline: 1.1375x; 1.0053x over previous
import functools

import jax
import jax.numpy as jnp
from jax.experimental import pallas as pl
from jax.experimental.pallas import tpu as pltpu

_B, _S, _D, _F, _C, _R = 2, 2048, 1024, 4096, 8, 4
_CR = _C * _R
_ALPHA = 1.0

_N_TOK = _B * _S
_F_TILE = 512
_S_TILE_B = 1024


def _softmax_pr(logits_ref):
    logits = logits_ref[...]
    m = jnp.max(logits, axis=1, keepdims=True)
    ex = jnp.exp(logits - m)
    probs = ex / jnp.sum(ex, axis=1, keepdims=True)
    e = (jax.lax.broadcasted_iota(jnp.int32, (_C, _CR), 1) // _R ==
         jax.lax.broadcasted_iota(jnp.int32, (_C, _CR), 0)).astype(jnp.float32)
    pr = jnp.dot(probs, e, preferred_element_type=jnp.float32)
    return probs, pr


def _layer1_kernel(logits_ref, x_ref, w1_ref, b1_ref, lb1_ref,
                   a1_ref, b1f_ref, w2_ref, a2_ref, b2f_ref,
                   h_ref, w2eff_ref):
    probs, pr = _softmax_pr(logits_ref)
    w1_eff = w1_ref[...] + _ALPHA * jnp.dot(
        b1f_ref[...] * pr, a1_ref[...], preferred_element_type=jnp.float32)
    bias1 = b1_ref[...] + _ALPHA * jnp.dot(
        probs, lb1_ref[...], preferred_element_type=jnp.float32)
    h = jax.lax.dot_general(
        x_ref[...], w1_eff, (((1,), (1,)), ((), ())),
        preferred_element_type=jnp.float32)
    h_ref[...] = jnp.maximum(h + bias1, 0.0).astype(jnp.bfloat16)
    w2eff_ref[...] = (w2_ref[...] + _ALPHA * jnp.dot(
        b2f_ref[...] * pr, a2_ref[...],
        preferred_element_type=jnp.float32)).astype(jnp.bfloat16)


def _layer2_kernel(logits_ref, h_ref, w2eff_ref, b2_ref, lb2_ref,
                   out_ref):
    probs, _ = _softmax_pr(logits_ref)
    bias2 = b2_ref[...] + _ALPHA * jnp.dot(
        probs, lb2_ref[...], preferred_element_type=jnp.float32)
    acc = jax.lax.dot_general(
        h_ref[...], w2eff_ref[...], (((1,), (1,)), ((), ())),
        preferred_element_type=jnp.float32)
    out_ref[...] = acc + bias2


@functools.partial(jax.jit, static_argnames=())
def kernel(x, W1, b1, W2, b2, router_logits, A1, B1, lb1, A2, B2, lb2):
    bsz, seq, d = x.shape
    xf = x.reshape(_N_TOK, d)

    a1f = A1.reshape(_CR, _D)
    b1f = B1.transpose(1, 0, 2).reshape(_F, _CR)
    a2f = A2.reshape(_CR, _F)
    b2f = B2.transpose(1, 0, 2).reshape(_D, _CR)

    logits2d = router_logits.reshape(1, _C)
    b1r = b1.reshape(1, _F)
    b2r = b2.reshape(1, _D)

    n_f = _F // _F_TILE
    h, w2eff = pl.pallas_call(
        _layer1_kernel,
        grid=(n_f,),
        in_specs=[
            pl.BlockSpec((1, _C), lambda j: (0, 0)),
            pl.BlockSpec((_N_TOK, _D), lambda j: (0, 0)),
            pl.BlockSpec((_F_TILE, _D), lambda j: (j, 0)),
            pl.BlockSpec((1, _F_TILE), lambda j: (0, j)),
            pl.BlockSpec((_C, _F_TILE), lambda j: (0, j)),
            pl.BlockSpec((_CR, _D), lambda j: (0, 0)),
            pl.BlockSpec((_F_TILE, _CR), lambda j: (j, 0)),
            pl.BlockSpec((_D, _F_TILE), lambda j: (0, j)),
            pl.BlockSpec((_CR, _F_TILE), lambda j: (0, j)),
            pl.BlockSpec((_D, _CR), lambda j: (0, 0)),
        ],
        out_specs=[
            pl.BlockSpec((_N_TOK, _F_TILE), lambda j: (0, j)),
            pl.BlockSpec((_D, _F_TILE), lambda j: (0, j)),
        ],
        out_shape=[
            jax.ShapeDtypeStruct((_N_TOK, _F), jnp.bfloat16),
            jax.ShapeDtypeStruct((_D, _F), jnp.bfloat16),
        ],
        compiler_params=pltpu.CompilerParams(
            vmem_limit_bytes=100 * 1024 * 1024),
    )(logits2d, xf, W1, b1r, lb1, a1f, b1f, W2, a2f, b2f)

    n_s = _N_TOK // _S_TILE_B
    out = pl.pallas_call(
        _layer2_kernel,
        grid=(n_s,),
        in_specs=[
            pl.BlockSpec((1, _C), lambda i: (0, 0)),
            pl.BlockSpec((_S_TILE_B, _F), lambda i: (i, 0)),
            pl.BlockSpec((_D, _F), lambda i: (0, 0)),
            pl.BlockSpec((1, _D), lambda i: (0, 0)),
            pl.BlockSpec((_C, _D), lambda i: (0, 0)),
        ],
        out_specs=pl.BlockSpec((_S_TILE_B, _D), lambda i: (i, 0)),
        out_shape=jax.ShapeDtypeStruct((_N_TOK, _D), jnp.float32),
        compiler_params=pltpu.CompilerParams(
            vmem_limit_bytes=100 * 1024 * 1024),
    )(logits2d, h, w2eff, b2r, lb2)

    return out.reshape(bsz, seq, d)
